# baseline (speedup 1.0000x reference)
import functools

import jax
import jax.numpy as jnp
from jax import lax
from jax.experimental import pallas as pl
from jax.experimental.pallas import tpu as pltpu
from jax.experimental.pallas import tpu_sc as plsc

N = 10000
D = 128
D_OUT = 64
E = 320000

NC = 2
NS = 16
NW = NC * NS
CHUNK = 125
CHUNKS = 80
GROUP = 40
N_PAD = 10240
ROWS_PER_TILE = N_PAD // NS


def _sc_scatter_build():
    mesh = plsc.VectorSubcoreMesh(core_axis_name="c", subcore_axis_name="s")

    @functools.partial(
        pl.kernel,
        mesh=mesh,
        out_type=jax.ShapeDtypeStruct((NC, N_PAD, D), jnp.float32),
        scratch_types=[
            pltpu.VMEM((GROUP, CHUNK), jnp.int32),
            pltpu.VMEM((GROUP, CHUNK), jnp.int32),
            pltpu.VMEM((CHUNK, D), jnp.float32),
            pltpu.VMEM((CHUNK, D), jnp.float32),
            pltpu.VMEM_SHARED((N_PAD, D), jnp.float32),
            pltpu.SemaphoreType.DMA,
            pltpu.SemaphoreType.DMA,
            pltpu.SemaphoreType.DMA,
            pltpu.SemaphoreType.DMA,
        ],
    )
    def sc_scatter(src_hbm, dst_hbm, x_hbm, out_hbm,
                   src_v, dst_v, rows_0, rows_1, acc_sh,
                   sem_0, sem_1, sem_s0, sem_s1):
        c = lax.axis_index("c")
        s = lax.axis_index("s")
        w = c * NS + s

        pltpu.sync_copy(x_hbm.at[pl.ds(s * ROWS_PER_TILE, ROWS_PER_TILE)],
                        acc_sh.at[pl.ds(s * ROWS_PER_TILE, ROWS_PER_TILE)])
        plsc.subcore_barrier()

        rows = (rows_0, rows_1)
        gsems = (sem_0, sem_1)
        ssems = (sem_s0, sem_s1)
        for h in range(CHUNKS // GROUP):
            pltpu.sync_copy(src_hbm.at[w, pl.ds(h * GROUP, GROUP)], src_v)
            pltpu.sync_copy(dst_hbm.at[w, pl.ds(h * GROUP, GROUP)], dst_v)

            pltpu.async_copy(x_hbm.at[src_v.at[0]], rows[0], gsems[0])

            def body2(jj, carry):
                j0 = jj * 2
                for b in range(2):
                    j = j0 + b
                    pltpu.make_async_copy(x_hbm.at[src_v.at[j]], rows[b],
                                          gsems[b]).wait()
                    pltpu.async_copy(rows[b], acc_sh.at[dst_v.at[j]],
                                     ssems[b], add=True)

                    @pl.when(j >= 1)
                    def _():
                        pltpu.make_async_copy(rows[1 - b],
                                              acc_sh.at[dst_v.at[j]],
                                              ssems[1 - b]).wait()

                    @pl.when(j + 1 < GROUP)
                    def _():
                        pltpu.async_copy(x_hbm.at[src_v.at[j + 1]],
                                         rows[1 - b], gsems[1 - b])

                return carry

            lax.fori_loop(0, GROUP // 2, body2, 0)
            pltpu.make_async_copy(rows[1], acc_sh.at[dst_v.at[0]],
                                  ssems[1]).wait()
        plsc.subcore_barrier()

        pltpu.sync_copy(acc_sh.at[pl.ds(s * ROWS_PER_TILE, ROWS_PER_TILE)],
                        out_hbm.at[c, pl.ds(s * ROWS_PER_TILE, ROWS_PER_TILE)])

    return sc_scatter


_sc_scatter = _sc_scatter_build()


def _mlp_mid_body(x_ref, p_ref, wa_ref, ba_ref, wb_ref, bb_ref, o_ref):
    t = p_ref[0] + p_ref[1] - x_ref[...]
    u = jnp.maximum(
        jnp.dot(t, wa_ref[...], preferred_element_type=jnp.float32)
        + ba_ref[...], 0.0)
    v = jnp.dot(u, wb_ref[...], preferred_element_type=jnp.float32) + bb_ref[...]
    o_ref[...] = jnp.maximum(v, 0.0)


def _mlp_last_body(x_ref, p_ref, wa_ref, ba_ref, wb_ref, bb_ref,
                   w3_ref, b3_ref, o_ref):
    t = p_ref[0] + p_ref[1] - x_ref[...]
    u = jnp.maximum(
        jnp.dot(t, wa_ref[...], preferred_element_type=jnp.float32)
        + ba_ref[...], 0.0)
    v = jnp.dot(u, wb_ref[...], preferred_element_type=jnp.float32) + bb_ref[...]
    h = jnp.maximum(v, 0.0)
    o_ref[...] = (jnp.dot(h, w3_ref[...], preferred_element_type=jnp.float32)
                  + b3_ref[...])


_RB = 1024


def _tc_mlp_mid(x, p, wa, ba, wb, bb):
    grid = (N_PAD // _RB,)
    return pl.pallas_call(
        _mlp_mid_body,
        grid=grid,
        in_specs=[
            pl.BlockSpec((_RB, D), lambda i: (i, 0)),
            pl.BlockSpec((NC, _RB, D), lambda i: (0, i, 0)),
            pl.BlockSpec((D, D), lambda i: (0, 0)),
            pl.BlockSpec((1, D), lambda i: (0, 0)),
            pl.BlockSpec((D, D), lambda i: (0, 0)),
            pl.BlockSpec((1, D), lambda i: (0, 0)),
        ],
        out_specs=pl.BlockSpec((_RB, D), lambda i: (i, 0)),
        out_shape=jax.ShapeDtypeStruct((N_PAD, D), jnp.float32),
    )(x, p, wa, ba, wb, bb)


def _tc_mlp_last(x, p, wa, ba, wb, bb, w3, b3):
    grid = (N // _RB + 1,)
    return pl.pallas_call(
        _mlp_last_body,
        grid=grid,
        in_specs=[
            pl.BlockSpec((_RB, D), lambda i: (i, 0)),
            pl.BlockSpec((NC, _RB, D), lambda i: (0, i, 0)),
            pl.BlockSpec((D, D), lambda i: (0, 0)),
            pl.BlockSpec((1, D), lambda i: (0, 0)),
            pl.BlockSpec((D, D), lambda i: (0, 0)),
            pl.BlockSpec((1, D), lambda i: (0, 0)),
            pl.BlockSpec((D, D_OUT), lambda i: (0, 0)),
            pl.BlockSpec((1, D_OUT), lambda i: (0, 0)),
        ],
        out_specs=pl.BlockSpec((_RB, D_OUT), lambda i: (i, 0)),
        out_shape=jax.ShapeDtypeStruct((N, D_OUT), jnp.float32),
    )(x, p, wa, ba, wb, bb, w3, b3)


def kernel(x, edge_index, W1a, b1a, W1b, b1b, W2a, b2a, W2b, b2b, W3, b3):
    src = edge_index[0].astype(jnp.int32)
    dst = edge_index[1].astype(jnp.int32)
    src_r = src.reshape(NW, CHUNKS, CHUNK)
    dst_r = dst.reshape(NW, CHUNKS, CHUNK)

    x_pad = jnp.concatenate([x, jnp.zeros((N_PAD - N, D), jnp.float32)])

    b1a2 = b1a.reshape(1, D)
    b1b2 = b1b.reshape(1, D)
    b2a2 = b2a.reshape(1, D)
    b2b2 = b2b.reshape(1, D)
    b32 = b3.reshape(1, D_OUT)

    p1 = _sc_scatter(src_r, dst_r, x_pad)
    h1 = _tc_mlp_mid(x_pad, p1, W1a, b1a2, W1b, b1b2)
    p2 = _sc_scatter(src_r, dst_r, h1)
    out = _tc_mlp_last(h1, p2, W2a, b2a2, W2b, b2b2, W3, b32)
    return out


# speedup vs baseline: 1.1530x; 1.1530x over previous
import functools

import jax
import jax.numpy as jnp
from jax import lax
from jax.experimental import pallas as pl
from jax.experimental.pallas import tpu as pltpu
from jax.experimental.pallas import tpu_sc as plsc

N = 10000
D = 128
D_OUT = 64
E = 320000

NC = 2
NS = 16
NW = NC * NS
CHUNK = 125
CHUNKS = 80
GROUP = 40
N_PAD = 10240
ROWS_PER_TILE = N_PAD // NS


def _sc_scatter_build():
    mesh = plsc.VectorSubcoreMesh(core_axis_name="c", subcore_axis_name="s")

    @functools.partial(
        pl.kernel,
        mesh=mesh,
        out_type=jax.ShapeDtypeStruct((NC, N_PAD, D), jnp.float32),
        scratch_types=[
            pltpu.VMEM((GROUP, CHUNK), jnp.int32),
            pltpu.VMEM((GROUP, CHUNK), jnp.int32),
            pltpu.VMEM((CHUNK, D), jnp.float32),
            pltpu.VMEM((CHUNK, D), jnp.float32),
            pltpu.VMEM_SHARED((N_PAD, D), jnp.float32),
            pltpu.SemaphoreType.DMA,
            pltpu.SemaphoreType.DMA,
        ],
    )
    def sc_scatter(src_hbm, dst_hbm, x_hbm, out_hbm,
                   src_v, dst_v, rows_0, rows_1, acc_sh, sem_0, sem_1):
        c = lax.axis_index("c")
        s = lax.axis_index("s")
        w = c * NS + s

        pltpu.sync_copy(x_hbm.at[pl.ds(s * ROWS_PER_TILE, ROWS_PER_TILE)],
                        acc_sh.at[pl.ds(s * ROWS_PER_TILE, ROWS_PER_TILE)])
        plsc.subcore_barrier()

        rows = (rows_0, rows_1)
        gsems = (sem_0, sem_1)
        for h in range(CHUNKS // GROUP):
            pltpu.sync_copy(src_hbm.at[w, pl.ds(h * GROUP, GROUP)], src_v)
            pltpu.sync_copy(dst_hbm.at[w, pl.ds(h * GROUP, GROUP)], dst_v)

            for b in range(2):
                pltpu.async_copy(x_hbm.at[src_v.at[b]], rows[b], gsems[b])

            def body2(jj, carry):
                j0 = jj * 2
                for b in range(2):
                    j = j0 + b
                    pltpu.make_async_copy(x_hbm.at[src_v.at[j]], rows[b],
                                          gsems[b]).wait()
                    pltpu.sync_copy(rows[b], acc_sh.at[dst_v.at[j]], add=True)

                    @pl.when(j + 2 < GROUP)
                    def _():
                        pltpu.async_copy(x_hbm.at[src_v.at[j + 2]], rows[b],
                                         gsems[b])

                return carry

            lax.fori_loop(0, GROUP // 2, body2, 0)
        plsc.subcore_barrier()

        pltpu.sync_copy(acc_sh.at[pl.ds(s * ROWS_PER_TILE, ROWS_PER_TILE)],
                        out_hbm.at[c, pl.ds(s * ROWS_PER_TILE, ROWS_PER_TILE)])

    return sc_scatter


_sc_scatter = _sc_scatter_build()


def _mlp_mid_body(x_ref, p_ref, wa_ref, ba_ref, wb_ref, bb_ref, o_ref):
    t = p_ref[0] + p_ref[1] - x_ref[...]
    u = jnp.maximum(
        jnp.dot(t, wa_ref[...], preferred_element_type=jnp.float32)
        + ba_ref[...], 0.0)
    v = jnp.dot(u, wb_ref[...], preferred_element_type=jnp.float32) + bb_ref[...]
    o_ref[...] = jnp.maximum(v, 0.0)


def _mlp_last_body(x_ref, p_ref, wa_ref, ba_ref, wb_ref, bb_ref,
                   w3_ref, b3_ref, o_ref):
    t = p_ref[0] + p_ref[1] - x_ref[...]
    u = jnp.maximum(
        jnp.dot(t, wa_ref[...], preferred_element_type=jnp.float32)
        + ba_ref[...], 0.0)
    v = jnp.dot(u, wb_ref[...], preferred_element_type=jnp.float32) + bb_ref[...]
    h = jnp.maximum(v, 0.0)
    o_ref[...] = (jnp.dot(h, w3_ref[...], preferred_element_type=jnp.float32)
                  + b3_ref[...])


_RB = 1024


def _tc_mlp_mid(x, p, wa, ba, wb, bb):
    grid = (N_PAD // _RB,)
    return pl.pallas_call(
        _mlp_mid_body,
        grid=grid,
        in_specs=[
            pl.BlockSpec((_RB, D), lambda i: (i, 0)),
            pl.BlockSpec((NC, _RB, D), lambda i: (0, i, 0)),
            pl.BlockSpec((D, D), lambda i: (0, 0)),
            pl.BlockSpec((1, D), lambda i: (0, 0)),
            pl.BlockSpec((D, D), lambda i: (0, 0)),
            pl.BlockSpec((1, D), lambda i: (0, 0)),
        ],
        out_specs=pl.BlockSpec((_RB, D), lambda i: (i, 0)),
        out_shape=jax.ShapeDtypeStruct((N_PAD, D), jnp.float32),
    )(x, p, wa, ba, wb, bb)


def _tc_mlp_last(x, p, wa, ba, wb, bb, w3, b3):
    grid = (N // _RB + 1,)
    return pl.pallas_call(
        _mlp_last_body,
        grid=grid,
        in_specs=[
            pl.BlockSpec((_RB, D), lambda i: (i, 0)),
            pl.BlockSpec((NC, _RB, D), lambda i: (0, i, 0)),
            pl.BlockSpec((D, D), lambda i: (0, 0)),
            pl.BlockSpec((1, D), lambda i: (0, 0)),
            pl.BlockSpec((D, D), lambda i: (0, 0)),
            pl.BlockSpec((1, D), lambda i: (0, 0)),
            pl.BlockSpec((D, D_OUT), lambda i: (0, 0)),
            pl.BlockSpec((1, D_OUT), lambda i: (0, 0)),
        ],
        out_specs=pl.BlockSpec((_RB, D_OUT), lambda i: (i, 0)),
        out_shape=jax.ShapeDtypeStruct((N, D_OUT), jnp.float32),
    )(x, p, wa, ba, wb, bb, w3, b3)


def kernel(x, edge_index, W1a, b1a, W1b, b1b, W2a, b2a, W2b, b2b, W3, b3):
    src = edge_index[0].astype(jnp.int32)
    dst = edge_index[1].astype(jnp.int32)
    src_r = src.reshape(NW, CHUNKS, CHUNK)
    dst_r = dst.reshape(NW, CHUNKS, CHUNK)

    x_pad = jnp.concatenate([x, jnp.zeros((N_PAD - N, D), jnp.float32)])

    b1a2 = b1a.reshape(1, D)
    b1b2 = b1b.reshape(1, D)
    b2a2 = b2a.reshape(1, D)
    b2b2 = b2b.reshape(1, D)
    b32 = b3.reshape(1, D_OUT)

    p1 = _sc_scatter(src_r, dst_r, x_pad)
    h1 = _tc_mlp_mid(x_pad, p1, W1a, b1a2, W1b, b1b2)
    p2 = _sc_scatter(src_r, dst_r, h1)
    out = _tc_mlp_last(h1, p2, W2a, b2a2, W2b, b2b2, W3, b32)
    return out


# speedup vs baseline: 1.1750x; 1.0191x over previous
import functools

import jax
import jax.numpy as jnp
from jax import lax
from jax.experimental import pallas as pl
from jax.experimental.pallas import tpu as pltpu
from jax.experimental.pallas import tpu_sc as plsc

N = 10000
D = 128
D_OUT = 64
E = 320000

NC = 2
NS = 16
NW = NC * NS
CHUNK = 125
CHUNKS = 80
GROUP = 40
N_PAD = 10240
ROWS_PER_TILE = N_PAD // NS
TAIL_ROWS = N - (NS - 1) * ROWS_PER_TILE


def _sc_scatter_build():
    mesh = plsc.VectorSubcoreMesh(core_axis_name="c", subcore_axis_name="s")

    @functools.partial(
        pl.kernel,
        mesh=mesh,
        out_type=jax.ShapeDtypeStruct((NC, N_PAD, D), jnp.float32),
        scratch_types=[
            pltpu.VMEM((GROUP, CHUNK), jnp.int32),
            pltpu.VMEM((GROUP, CHUNK), jnp.int32),
            pltpu.VMEM((CHUNK, D), jnp.float32),
            pltpu.VMEM((CHUNK, D), jnp.float32),
            pltpu.VMEM_SHARED((N_PAD, D), jnp.float32),
            pltpu.SemaphoreType.DMA,
            pltpu.SemaphoreType.DMA,
        ],
    )
    def sc_scatter(src_hbm, dst_hbm, x_hbm, out_hbm,
                   src_v, dst_v, rows_0, rows_1, acc_sh, sem_0, sem_1):
        c = lax.axis_index("c")
        s = lax.axis_index("s")
        w = c * NS + s

        @pl.when(s < NS - 1)
        def _():
            pltpu.sync_copy(x_hbm.at[pl.ds(s * ROWS_PER_TILE, ROWS_PER_TILE)],
                            acc_sh.at[pl.ds(s * ROWS_PER_TILE, ROWS_PER_TILE)])

        @pl.when(s == NS - 1)
        def _():
            pltpu.sync_copy(x_hbm.at[pl.ds(N - TAIL_ROWS, TAIL_ROWS)],
                            acc_sh.at[pl.ds(N - TAIL_ROWS, TAIL_ROWS)])

        plsc.subcore_barrier()

        rows = (rows_0, rows_1)
        gsems = (sem_0, sem_1)
        for h in range(CHUNKS // GROUP):
            pltpu.sync_copy(src_hbm.at[w, pl.ds(h * GROUP, GROUP)], src_v)
            pltpu.sync_copy(dst_hbm.at[w, pl.ds(h * GROUP, GROUP)], dst_v)

            for b in range(2):
                pltpu.async_copy(x_hbm.at[src_v.at[b]], rows[b], gsems[b])

            def body2(jj, carry):
                j0 = jj * 2
                for b in range(2):
                    j = j0 + b
                    pltpu.make_async_copy(x_hbm.at[src_v.at[j]], rows[b],
                                          gsems[b]).wait()
                    pltpu.sync_copy(rows[b], acc_sh.at[dst_v.at[j]], add=True)

                    @pl.when(j + 2 < GROUP)
                    def _():
                        pltpu.async_copy(x_hbm.at[src_v.at[j + 2]], rows[b],
                                         gsems[b])

                return carry

            lax.fori_loop(0, GROUP // 2, body2, 0)

        plsc.subcore_barrier()

        pltpu.sync_copy(acc_sh.at[pl.ds(s * ROWS_PER_TILE, ROWS_PER_TILE)],
                        out_hbm.at[c, pl.ds(s * ROWS_PER_TILE, ROWS_PER_TILE)])

    return sc_scatter


_sc_scatter = _sc_scatter_build()


def _mlp_mid_body(x_ref, p_ref, wa_ref, ba_ref, wb_ref, bb_ref, o_ref):
    t = p_ref[0] + p_ref[1] - x_ref[...]
    u = jnp.maximum(
        jnp.dot(t, wa_ref[...], preferred_element_type=jnp.float32)
        + ba_ref[...], 0.0)
    v = jnp.dot(u, wb_ref[...], preferred_element_type=jnp.float32) + bb_ref[...]
    o_ref[...] = jnp.maximum(v, 0.0)


def _mlp_last_body(x_ref, p_ref, wa_ref, ba_ref, wb_ref, bb_ref,
                   w3_ref, b3_ref, o_ref):
    t = p_ref[0] + p_ref[1] - x_ref[...]
    u = jnp.maximum(
        jnp.dot(t, wa_ref[...], preferred_element_type=jnp.float32)
        + ba_ref[...], 0.0)
    v = jnp.dot(u, wb_ref[...], preferred_element_type=jnp.float32) + bb_ref[...]
    h = jnp.maximum(v, 0.0)
    o_ref[...] = (jnp.dot(h, w3_ref[...], preferred_element_type=jnp.float32)
                  + b3_ref[...])


_RB = 1024


def _tc_mlp_mid(x, p, wa, ba, wb, bb):
    grid = (N // _RB + 1,)
    return pl.pallas_call(
        _mlp_mid_body,
        grid=grid,
        in_specs=[
            pl.BlockSpec((_RB, D), lambda i: (i, 0)),
            pl.BlockSpec((NC, _RB, D), lambda i: (0, i, 0)),
            pl.BlockSpec((D, D), lambda i: (0, 0)),
            pl.BlockSpec((1, D), lambda i: (0, 0)),
            pl.BlockSpec((D, D), lambda i: (0, 0)),
            pl.BlockSpec((1, D), lambda i: (0, 0)),
        ],
        out_specs=pl.BlockSpec((_RB, D), lambda i: (i, 0)),
        out_shape=jax.ShapeDtypeStruct((N, D), jnp.float32),
    )(x, p, wa, ba, wb, bb)


def _tc_mlp_last(x, p, wa, ba, wb, bb, w3, b3):
    grid = (N // _RB + 1,)
    return pl.pallas_call(
        _mlp_last_body,
        grid=grid,
        in_specs=[
            pl.BlockSpec((_RB, D), lambda i: (i, 0)),
            pl.BlockSpec((NC, _RB, D), lambda i: (0, i, 0)),
            pl.BlockSpec((D, D), lambda i: (0, 0)),
            pl.BlockSpec((1, D), lambda i: (0, 0)),
            pl.BlockSpec((D, D), lambda i: (0, 0)),
            pl.BlockSpec((1, D), lambda i: (0, 0)),
            pl.BlockSpec((D, D_OUT), lambda i: (0, 0)),
            pl.BlockSpec((1, D_OUT), lambda i: (0, 0)),
        ],
        out_specs=pl.BlockSpec((_RB, D_OUT), lambda i: (i, 0)),
        out_shape=jax.ShapeDtypeStruct((N, D_OUT), jnp.float32),
    )(x, p, wa, ba, wb, bb, w3, b3)


def kernel(x, edge_index, W1a, b1a, W1b, b1b, W2a, b2a, W2b, b2b, W3, b3):
    src = edge_index[0].astype(jnp.int32)
    dst = edge_index[1].astype(jnp.int32)
    src_r = src.reshape(NW, CHUNKS, CHUNK)
    dst_r = dst.reshape(NW, CHUNKS, CHUNK)

    b1a2 = b1a.reshape(1, D)
    b1b2 = b1b.reshape(1, D)
    b2a2 = b2a.reshape(1, D)
    b2b2 = b2b.reshape(1, D)
    b32 = b3.reshape(1, D_OUT)

    p1 = _sc_scatter(src_r, dst_r, x)
    h1 = _tc_mlp_mid(x, p1, W1a, b1a2, W1b, b1b2)
    p2 = _sc_scatter(src_r, dst_r, h1)
    out = _tc_mlp_last(h1, p2, W2a, b2a2, W2b, b2b2, W3, b32)
    return out


# speedup vs baseline: 1.1951x; 1.0171x over previous
import functools

import jax
import jax.numpy as jnp
from jax import lax
from jax.experimental import pallas as pl
from jax.experimental.pallas import tpu as pltpu
from jax.experimental.pallas import tpu_sc as plsc

N = 10000
D = 128
D_OUT = 64
E = 320000

NC = 2
NS = 16
NW = NC * NS
CHUNK = 125
CHUNKS = 80
GROUP = 40
N_PAD = 10240
ROWS_PER_TILE = N_PAD // NS
TAIL_ROWS = N - (NS - 1) * ROWS_PER_TILE


def _sc_scatter_build():
    mesh = plsc.VectorSubcoreMesh(core_axis_name="c", subcore_axis_name="s")

    @functools.partial(
        pl.kernel,
        mesh=mesh,
        out_type=jax.ShapeDtypeStruct((NC, N_PAD, D), jnp.float32),
        scratch_types=[
            pltpu.VMEM((GROUP, CHUNK), jnp.int32),
            pltpu.VMEM((GROUP, CHUNK), jnp.int32),
            pltpu.VMEM((CHUNK, D), jnp.float32),
            pltpu.VMEM((CHUNK, D), jnp.float32),
            pltpu.VMEM_SHARED((N_PAD, D), jnp.float32),
            pltpu.SemaphoreType.DMA,
            pltpu.SemaphoreType.DMA,
        ],
    )
    def sc_scatter(src_hbm, dst_hbm, x_hbm, out_hbm,
                   src_v, dst_v, rows_0, rows_1, acc_sh, sem_0, sem_1):
        c = lax.axis_index("c")
        s = lax.axis_index("s")
        w = c * NS + s

        @pl.when(s < NS - 1)
        def _():
            pltpu.sync_copy(x_hbm.at[pl.ds(s * ROWS_PER_TILE, ROWS_PER_TILE)],
                            acc_sh.at[pl.ds(s * ROWS_PER_TILE, ROWS_PER_TILE)])

        @pl.when(s == NS - 1)
        def _():
            pltpu.sync_copy(x_hbm.at[pl.ds(N - TAIL_ROWS, TAIL_ROWS)],
                            acc_sh.at[pl.ds(N - TAIL_ROWS, TAIL_ROWS)])

        plsc.subcore_barrier()

        rows = (rows_0, rows_1)
        gsems = (sem_0, sem_1)
        for h in range(CHUNKS // GROUP):
            pltpu.sync_copy(src_hbm.at[w, pl.ds(h * GROUP, GROUP)], src_v)
            pltpu.sync_copy(dst_hbm.at[w, pl.ds(h * GROUP, GROUP)], dst_v)

            for b in range(2):
                pltpu.async_copy(x_hbm.at[src_v.at[b]], rows[b], gsems[b])

            def body2(jj, carry):
                j0 = jj * 2
                for b in range(2):
                    j = j0 + b
                    pltpu.make_async_copy(x_hbm.at[src_v.at[j]], rows[b],
                                          gsems[b]).wait()
                    pltpu.sync_copy(rows[b], acc_sh.at[dst_v.at[j]], add=True)

                    @pl.when(j + 2 < GROUP)
                    def _():
                        pltpu.async_copy(x_hbm.at[src_v.at[j + 2]], rows[b],
                                         gsems[b])

                return carry

            lax.fori_loop(0, GROUP // 2, body2, 0)

        plsc.subcore_barrier()

        pltpu.sync_copy(acc_sh.at[pl.ds(s * ROWS_PER_TILE, ROWS_PER_TILE)],
                        out_hbm.at[c, pl.ds(s * ROWS_PER_TILE, ROWS_PER_TILE)])

    return sc_scatter


_sc_scatter = _sc_scatter_build()


def _mlp_mid_body(x_ref, p_ref, wa_ref, ba_ref, wb_ref, bb_ref, o_ref):
    t = p_ref[0] + p_ref[1] - x_ref[...]
    u = jnp.maximum(
        jnp.dot(t, wa_ref[...], preferred_element_type=jnp.float32)
        + ba_ref[...], 0.0)
    v = jnp.dot(u, wb_ref[...], preferred_element_type=jnp.float32) + bb_ref[...]
    o_ref[...] = jnp.maximum(v, 0.0)


def _mlp_last_body(x_ref, p_ref, wa_ref, ba_ref, wb_ref, bb_ref,
                   w3_ref, b3_ref, o_ref):
    t = p_ref[0] + p_ref[1] - x_ref[...]
    u = jnp.maximum(
        jnp.dot(t, wa_ref[...], preferred_element_type=jnp.float32)
        + ba_ref[...], 0.0)
    v = jnp.dot(u, wb_ref[...], preferred_element_type=jnp.float32) + bb_ref[...]
    h = jnp.maximum(v, 0.0)
    o_ref[...] = (jnp.dot(h, w3_ref[...], preferred_element_type=jnp.float32)
                  + b3_ref[...])


_RB = 2048


def _tc_mlp_mid(x, p, wa, ba, wb, bb):
    grid = (N // _RB + 1,)
    return pl.pallas_call(
        _mlp_mid_body,
        grid=grid,
        in_specs=[
            pl.BlockSpec((_RB, D), lambda i: (i, 0)),
            pl.BlockSpec((NC, _RB, D), lambda i: (0, i, 0)),
            pl.BlockSpec((D, D), lambda i: (0, 0)),
            pl.BlockSpec((1, D), lambda i: (0, 0)),
            pl.BlockSpec((D, D), lambda i: (0, 0)),
            pl.BlockSpec((1, D), lambda i: (0, 0)),
        ],
        out_specs=pl.BlockSpec((_RB, D), lambda i: (i, 0)),
        out_shape=jax.ShapeDtypeStruct((N, D), jnp.float32),
    )(x, p, wa, ba, wb, bb)


def _tc_mlp_last(x, p, wa, ba, wb, bb, w3, b3):
    grid = (N // _RB + 1,)
    return pl.pallas_call(
        _mlp_last_body,
        grid=grid,
        in_specs=[
            pl.BlockSpec((_RB, D), lambda i: (i, 0)),
            pl.BlockSpec((NC, _RB, D), lambda i: (0, i, 0)),
            pl.BlockSpec((D, D), lambda i: (0, 0)),
            pl.BlockSpec((1, D), lambda i: (0, 0)),
            pl.BlockSpec((D, D), lambda i: (0, 0)),
            pl.BlockSpec((1, D), lambda i: (0, 0)),
            pl.BlockSpec((D, D_OUT), lambda i: (0, 0)),
            pl.BlockSpec((1, D_OUT), lambda i: (0, 0)),
        ],
        out_specs=pl.BlockSpec((_RB, D_OUT), lambda i: (i, 0)),
        out_shape=jax.ShapeDtypeStruct((N, D_OUT), jnp.float32),
    )(x, p, wa, ba, wb, bb, w3, b3)


def kernel(x, edge_index, W1a, b1a, W1b, b1b, W2a, b2a, W2b, b2b, W3, b3):
    src = edge_index[0].astype(jnp.int32)
    dst = edge_index[1].astype(jnp.int32)
    src_r = src.reshape(NW, CHUNKS, CHUNK)
    dst_r = dst.reshape(NW, CHUNKS, CHUNK)

    b1a2 = b1a.reshape(1, D)
    b1b2 = b1b.reshape(1, D)
    b2a2 = b2a.reshape(1, D)
    b2b2 = b2b.reshape(1, D)
    b32 = b3.reshape(1, D_OUT)

    p1 = _sc_scatter(src_r, dst_r, x)
    h1 = _tc_mlp_mid(x, p1, W1a, b1a2, W1b, b1b2)
    p2 = _sc_scatter(src_r, dst_r, h1)
    out = _tc_mlp_last(h1, p2, W2a, b2a2, W2b, b2b2, W3, b32)
    return out


# speedup vs baseline: 1.2178x; 1.0190x over previous
import functools

import jax
import jax.numpy as jnp
from jax import lax
from jax.experimental import pallas as pl
from jax.experimental.pallas import tpu as pltpu
from jax.experimental.pallas import tpu_sc as plsc

N = 10000
D = 128
D_OUT = 64
E = 320000

NC = 2
NS = 16
NW = NC * NS
CHUNK = 125
CHUNKS = 80
GROUP = 40
N_PAD = 10240
ROWS_PER_TILE = N_PAD // NS
TAIL_ROWS = N - (NS - 1) * ROWS_PER_TILE


def _sc_scatter_build():
    mesh = plsc.VectorSubcoreMesh(core_axis_name="c", subcore_axis_name="s")

    @functools.partial(
        pl.kernel,
        mesh=mesh,
        out_type=jax.ShapeDtypeStruct((NC, N_PAD, D), jnp.float32),
        scratch_types=[
            pltpu.VMEM((GROUP, CHUNK), jnp.int32),
            pltpu.VMEM((GROUP, CHUNK), jnp.int32),
            pltpu.VMEM((CHUNK, D), jnp.float32),
            pltpu.VMEM((CHUNK, D), jnp.float32),
            pltpu.VMEM_SHARED((N_PAD, D), jnp.float32),
            pltpu.SemaphoreType.DMA,
            pltpu.SemaphoreType.DMA,
        ],
    )
    def sc_scatter(src_hbm, dst_hbm, x_hbm, out_hbm,
                   src_v, dst_v, rows_0, rows_1, acc_sh, sem_0, sem_1):
        c = lax.axis_index("c")
        s = lax.axis_index("s")
        w = c * NS + s

        rows = (rows_0, rows_1)
        gsems = (sem_0, sem_1)

        pltpu.sync_copy(src_hbm.at[w, pl.ds(0, GROUP)], src_v)
        pltpu.sync_copy(dst_hbm.at[w, pl.ds(0, GROUP)], dst_v)
        for b in range(2):
            pltpu.async_copy(x_hbm.at[src_v.at[b]], rows[b], gsems[b])

        @pl.when(s < NS - 1)
        def _():
            pltpu.sync_copy(x_hbm.at[pl.ds(s * ROWS_PER_TILE, ROWS_PER_TILE)],
                            acc_sh.at[pl.ds(s * ROWS_PER_TILE, ROWS_PER_TILE)])

        @pl.when(s == NS - 1)
        def _():
            pltpu.sync_copy(x_hbm.at[pl.ds(N - TAIL_ROWS, TAIL_ROWS)],
                            acc_sh.at[pl.ds(N - TAIL_ROWS, TAIL_ROWS)])

        plsc.subcore_barrier()

        for h in range(CHUNKS // GROUP):
            if h > 0:
                pltpu.sync_copy(src_hbm.at[w, pl.ds(h * GROUP, GROUP)], src_v)
                pltpu.sync_copy(dst_hbm.at[w, pl.ds(h * GROUP, GROUP)], dst_v)

                for b in range(2):
                    pltpu.async_copy(x_hbm.at[src_v.at[b]], rows[b], gsems[b])

            def body2(jj, carry):
                j0 = jj * 2
                for b in range(2):
                    j = j0 + b
                    pltpu.make_async_copy(x_hbm.at[src_v.at[j]], rows[b],
                                          gsems[b]).wait()
                    pltpu.sync_copy(rows[b], acc_sh.at[dst_v.at[j]], add=True)

                    @pl.when(j + 2 < GROUP)
                    def _():
                        pltpu.async_copy(x_hbm.at[src_v.at[j + 2]], rows[b],
                                         gsems[b])

                return carry

            lax.fori_loop(0, GROUP // 2, body2, 0)

        plsc.subcore_barrier()

        pltpu.sync_copy(acc_sh.at[pl.ds(s * ROWS_PER_TILE, ROWS_PER_TILE)],
                        out_hbm.at[c, pl.ds(s * ROWS_PER_TILE, ROWS_PER_TILE)])

    return sc_scatter


_sc_scatter = _sc_scatter_build()


def _mlp_mid_body(x_ref, p_ref, wa_ref, ba_ref, wb_ref, bb_ref, o_ref):
    t = p_ref[0] + p_ref[1] - x_ref[...]
    u = jnp.maximum(
        jnp.dot(t, wa_ref[...], preferred_element_type=jnp.float32)
        + ba_ref[...], 0.0)
    v = jnp.dot(u, wb_ref[...], preferred_element_type=jnp.float32) + bb_ref[...]
    o_ref[...] = jnp.maximum(v, 0.0)


def _mlp_last_body(x_ref, p_ref, wa_ref, ba_ref, wb_ref, bb_ref,
                   w3_ref, b3_ref, o_ref):
    t = p_ref[0] + p_ref[1] - x_ref[...]
    u = jnp.maximum(
        jnp.dot(t, wa_ref[...], preferred_element_type=jnp.float32)
        + ba_ref[...], 0.0)
    v = jnp.dot(u, wb_ref[...], preferred_element_type=jnp.float32) + bb_ref[...]
    h = jnp.maximum(v, 0.0)
    o_ref[...] = (jnp.dot(h, w3_ref[...], preferred_element_type=jnp.float32)
                  + b3_ref[...])


_RB = 2560


def _tc_mlp_mid(x, p, wa, ba, wb, bb):
    grid = (N // _RB + 1,)
    return pl.pallas_call(
        _mlp_mid_body,
        grid=grid,
        in_specs=[
            pl.BlockSpec((_RB, D), lambda i: (i, 0)),
            pl.BlockSpec((NC, _RB, D), lambda i: (0, i, 0)),
            pl.BlockSpec((D, D), lambda i: (0, 0)),
            pl.BlockSpec((1, D), lambda i: (0, 0)),
            pl.BlockSpec((D, D), lambda i: (0, 0)),
            pl.BlockSpec((1, D), lambda i: (0, 0)),
        ],
        out_specs=pl.BlockSpec((_RB, D), lambda i: (i, 0)),
        out_shape=jax.ShapeDtypeStruct((N, D), jnp.float32),
    )(x, p, wa, ba, wb, bb)


def _tc_mlp_last(x, p, wa, ba, wb, bb, w3, b3):
    grid = (N // _RB + 1,)
    return pl.pallas_call(
        _mlp_last_body,
        grid=grid,
        in_specs=[
            pl.BlockSpec((_RB, D), lambda i: (i, 0)),
            pl.BlockSpec((NC, _RB, D), lambda i: (0, i, 0)),
            pl.BlockSpec((D, D), lambda i: (0, 0)),
            pl.BlockSpec((1, D), lambda i: (0, 0)),
            pl.BlockSpec((D, D), lambda i: (0, 0)),
            pl.BlockSpec((1, D), lambda i: (0, 0)),
            pl.BlockSpec((D, D_OUT), lambda i: (0, 0)),
            pl.BlockSpec((1, D_OUT), lambda i: (0, 0)),
        ],
        out_specs=pl.BlockSpec((_RB, D_OUT), lambda i: (i, 0)),
        out_shape=jax.ShapeDtypeStruct((N, D_OUT), jnp.float32),
    )(x, p, wa, ba, wb, bb, w3, b3)


def kernel(x, edge_index, W1a, b1a, W1b, b1b, W2a, b2a, W2b, b2b, W3, b3):
    src = edge_index[0].astype(jnp.int32)
    dst = edge_index[1].astype(jnp.int32)
    src_r = src.reshape(NW, CHUNKS, CHUNK)
    dst_r = dst.reshape(NW, CHUNKS, CHUNK)

    b1a2 = b1a.reshape(1, D)
    b1b2 = b1b.reshape(1, D)
    b2a2 = b2a.reshape(1, D)
    b2b2 = b2b.reshape(1, D)
    b32 = b3.reshape(1, D_OUT)

    p1 = _sc_scatter(src_r, dst_r, x)
    h1 = _tc_mlp_mid(x, p1, W1a, b1a2, W1b, b1b2)
    p2 = _sc_scatter(src_r, dst_r, h1)
    out = _tc_mlp_last(h1, p2, W2a, b2a2, W2b, b2b2, W3, b32)
    return out
